# Initial kernel scaffold; baseline (speedup 1.0000x reference)
import functools

import jax
import jax.numpy as jnp
from jax import lax
from jax.experimental import pallas as pl
from jax.experimental.pallas import tpu as pltpu
from jax.experimental.pallas import tpu_sc as plsc

B = 4
S = 4096
DIM = 768
N = B * S
NC = 2
NS = 16
NW = NC * NS
PER_W = N // NW
R = 128
CHUNKS = PER_W // R

_mesh = plsc.VectorSubcoreMesh(
    core_axis_name="c", subcore_axis_name="s", num_cores=NC, num_subcores=NS
)


@functools.partial(
    pl.kernel,
    out_type=jax.ShapeDtypeStruct((N, DIM), jnp.float32),
    mesh=_mesh,
    scratch_types=[
        pltpu.VMEM((R,), jnp.int32),
        pltpu.VMEM((R, DIM), jnp.float32),
        pltpu.SemaphoreType.DMA,
    ],
)
def _embed(idx_hbm, emb_hbm, pos_hbm, out_hbm, idx_v, rows_v, sem):
    wid = lax.axis_index("s") * NC + lax.axis_index("c")
    base = wid * PER_W
    pos_base = lax.rem(base, S)

    def body(c, carry):
        off = c * R
        pltpu.sync_copy(idx_hbm.at[pl.ds(base + off, R)], idx_v)
        pltpu.sync_copy(pos_hbm.at[pl.ds(pos_base + off, R)], rows_v)
        pltpu.async_copy(emb_hbm.at[idx_v], rows_v, sem, add=True).wait()
        pltpu.sync_copy(rows_v, out_hbm.at[pl.ds(base + off, R)])
        return carry

    lax.fori_loop(0, CHUNKS, body, 0)


def kernel(x, emb_table, pos_table):
    out = _embed(x.reshape(N), emb_table, pos_table)
    return out.reshape(B, S, DIM)


# speedup vs baseline: 1.6034x; 1.6034x over previous
import functools

import jax
import jax.numpy as jnp
from jax import lax
from jax.experimental import pallas as pl
from jax.experimental.pallas import tpu as pltpu
from jax.experimental.pallas import tpu_sc as plsc

B = 4
S = 4096
DIM = 768
N = B * S
NC = 2
NS = 16
NW = NC * NS
POS_W = S // NW
R = 64
PCHUNKS = POS_W // R
LANES = 16
VECS = DIM // LANES

_mesh = plsc.VectorSubcoreMesh(
    core_axis_name="c", subcore_axis_name="s", num_cores=NC, num_subcores=NS
)


@functools.partial(
    pl.kernel,
    out_type=jax.ShapeDtypeStruct((N, DIM), jnp.float32),
    mesh=_mesh,
    scratch_types=[
        pltpu.VMEM((R,), jnp.int32),
        pltpu.VMEM((R, DIM), jnp.float32),
        pltpu.VMEM((R, DIM), jnp.float32),
        pltpu.SemaphoreType.DMA,
    ],
)
def _embed(idx_hbm, emb_hbm, pos_hbm, out_hbm, idx_v, pos_v, rows_v, sem):
    wid = lax.axis_index("s") * NC + lax.axis_index("c")
    pos_base = wid * POS_W

    for pc in range(PCHUNKS):
        p0 = pos_base + pc * R
        pltpu.sync_copy(pos_hbm.at[pl.ds(p0, R)], pos_v)
        for b in range(B):
            base = b * S + p0
            pltpu.sync_copy(idx_hbm.at[pl.ds(base, R)], idx_v)
            pltpu.async_copy(emb_hbm.at[idx_v], rows_v, sem).wait()

            def add_row(r, carry):
                for j in range(VECS):
                    sl = pl.ds(j * LANES, LANES)
                    plsc.addupdate(rows_v.at[r, sl], pos_v[r, sl])
                return carry

            lax.fori_loop(0, R, add_row, 0)
            pltpu.sync_copy(rows_v, out_hbm.at[pl.ds(base, R)])


def kernel(x, emb_table, pos_table):
    out = _embed(x.reshape(N), emb_table, pos_table)
    return out.reshape(B, S, DIM)
